# baseline (speedup 1.0000x reference)
import functools

import jax
import jax.numpy as jnp
from jax import lax
from jax.experimental import pallas as pl
from jax.experimental.pallas import tpu as pltpu
from jax.experimental.pallas import tpu_sc as plsc

N = 4096
D = 1024
H = 4096
E = 16
K = 2
LANES = 128
BLK = 256
P = N * K + E * BLK
R = P // BLK
GB = 512


def _gating_kernel(lg_ref, idx_ref, w_ref, stats_ref, loss_ref, rank_ref):
    r = pl.program_id(0)
    nblk = pl.num_programs(0)
    logits = lg_ref[...]
    lane = lax.broadcasted_iota(jnp.int32, (GB, LANES), 1)
    valid = lane < E
    neg = jnp.float32(-1e30)
    lg = jnp.where(valid, logits, neg)
    m1 = jnp.max(lg, axis=1, keepdims=True)
    is1 = (lg == m1) & valid
    a1 = jnp.min(jnp.where(is1, lane, LANES), axis=1, keepdims=True)
    lg2 = jnp.where(lane == a1, neg, lg)
    m2 = jnp.max(lg2, axis=1, keepdims=True)
    is2 = (lg2 == m2) & valid
    a2 = jnp.min(jnp.where(is2, lane, LANES), axis=1, keepdims=True)
    w1 = 1.0 / (1.0 + jnp.exp(m2 - m1))
    w2 = 1.0 - w1
    idx_ref[...] = jnp.where(lane == 0, a1, jnp.where(lane == 1, a2, 0))
    w_ref[...] = jnp.where(lane == 0, w1, jnp.where(lane == 1, w2, 0.0))
    p = jnp.where(valid, jnp.exp(lg - m1), 0.0)
    p = p / jnp.sum(p, axis=1, keepdims=True)
    onehot = ((lane == a1) | (lane == a2)).astype(jnp.float32)

    @pl.when(r == 0)
    def _():
        stats_ref[...] = jnp.zeros_like(stats_ref)

    oh1 = (lane == a1).astype(jnp.float32)
    oh2 = (lane == a2).astype(jnp.float32)
    ri = lax.broadcasted_iota(jnp.int32, (GB, GB), 0)
    ci = lax.broadcasted_iota(jnp.int32, (GB, GB), 1)
    lt = (ri > ci).astype(jnp.float32)
    cum_excl = jnp.dot(lt, oh1 + oh2, preferred_element_type=jnp.float32)
    base = stats_ref[1:2, :]
    rank1 = jnp.sum((cum_excl + base) * oh1, axis=1, keepdims=True)
    rank2 = jnp.sum((cum_excl + base) * oh2, axis=1, keepdims=True)
    rank_ref[...] = jnp.where(lane == 0, rank1,
                              jnp.where(lane == 1, rank2, 0.0)).astype(
                                  jnp.int32)

    stats_ref[0:1, :] += jnp.sum(p, axis=0, keepdims=True)
    stats_ref[1:2, :] += jnp.sum(onehot, axis=0, keepdims=True)

    @pl.when(r == nblk - 1)
    def _():
        prob = stats_ref[0:1, :] * (1.0 / N)
        frac = stats_ref[1:2, :] * (1.0 / N)
        lb = E * jnp.sum(prob * frac)
        loss_ref[...] = jnp.full((1, LANES), lb, jnp.float32)


def _gating(logits):
    lgp = jnp.zeros((N, LANES), jnp.float32).at[:, :E].set(logits)
    nblk = N // GB
    return pl.pallas_call(
        _gating_kernel,
        grid=(nblk,),
        in_specs=[
            pl.BlockSpec((GB, LANES), lambda r: (r, 0)),
        ],
        out_specs=[
            pl.BlockSpec((GB, LANES), lambda r: (r, 0)),
            pl.BlockSpec((GB, LANES), lambda r: (r, 0)),
            pl.BlockSpec((2, LANES), lambda r: (0, 0)),
            pl.BlockSpec((1, LANES), lambda r: (0, 0)),
            pl.BlockSpec((GB, LANES), lambda r: (r, 0)),
        ],
        out_shape=[
            jax.ShapeDtypeStruct((N, LANES), jnp.int32),
            jax.ShapeDtypeStruct((N, LANES), jnp.float32),
            jax.ShapeDtypeStruct((2, LANES), jnp.float32),
            jax.ShapeDtypeStruct((1, LANES), jnp.float32),
            jax.ShapeDtypeStruct((N, LANES), jnp.int32),
        ],
    )(lgp)


def _sc_gather(table, idx):
    n_rows = idx.shape[0]
    d = table.shape[1]
    dt = table.dtype
    nw = 32
    b_per_w = n_rows // nw
    ch = 32
    n_ch = b_per_w // ch
    mesh = plsc.VectorSubcoreMesh(core_axis_name="c", subcore_axis_name="s")

    @functools.partial(
        pl.kernel,
        mesh=mesh,
        out_type=jax.ShapeDtypeStruct((n_rows, d), dt),
        scratch_types=[
            pltpu.VMEM((b_per_w,), jnp.int32),
            pltpu.VMEM((ch, d), dt),
            pltpu.VMEM((ch, d), dt),
            pltpu.SemaphoreType.DMA,
            pltpu.SemaphoreType.DMA,
        ],
    )
    def k(table_hbm, idx_hbm, out_hbm, idx_v, buf0, buf1, sem0, sem1):
        wid = lax.axis_index("s") * 2 + lax.axis_index("c")
        base = wid * b_per_w
        bufs = (buf0, buf1)
        sems = (sem0, sem1)
        pltpu.sync_copy(idx_hbm.at[pl.ds(base, b_per_w)], idx_v)
        pend = [None, None]
        pend[0] = pltpu.async_copy(
            table_hbm.at[idx_v.at[pl.ds(0, ch)]], bufs[0], sems[0])
        for c in range(n_ch):
            b = c % 2
            if c + 1 < n_ch:
                pend[1 - b] = pltpu.async_copy(
                    table_hbm.at[idx_v.at[pl.ds((c + 1) * ch, ch)]],
                    bufs[1 - b], sems[1 - b])
            pend[b].wait()
            pltpu.sync_copy(bufs[b], out_hbm.at[pl.ds(base + c * ch, ch)])

    return k(table, idx)


def _ffn_a_kernel(meta_ref, x_ref, w1_ref, b1_ref, h_ref):
    r = pl.program_id(0)

    @pl.when(r < meta_ref[R])
    def _():
        h = jnp.dot(x_ref[...], w1_ref[0],
                    preferred_element_type=jnp.float32) + b1_ref[0, 0][None, :]
        h_ref[...] = jnp.maximum(h, 0.0).astype(jnp.bfloat16)


def _ffn_a(x_pad, W1, b1, meta):
    grid_spec = pltpu.PrefetchScalarGridSpec(
        num_scalar_prefetch=1,
        grid=(R,),
        in_specs=[
            pl.BlockSpec((BLK, D), lambda r, m: (jnp.minimum(r, m[R] - 1), 0)),
            pl.BlockSpec((1, D, H), lambda r, m: (m[r], 0, 0)),
            pl.BlockSpec((1, 1, H), lambda r, m: (m[r], 0, 0)),
        ],
        out_specs=pl.BlockSpec(
            (BLK, H), lambda r, m: (jnp.minimum(r, m[R] - 1), 0)),
    )
    return pl.pallas_call(
        _ffn_a_kernel,
        grid_spec=grid_spec,
        out_shape=jax.ShapeDtypeStruct((P, H), jnp.bfloat16),
    )(meta, x_pad, W1, b1.reshape(E, 1, H))


def _ffn_b_kernel(meta_ref, h_ref, w2_ref, b2_ref, y_ref):
    r = pl.program_id(0)

    @pl.when(r < meta_ref[R])
    def _():
        y = jnp.dot(h_ref[...].astype(jnp.float32), w2_ref[0],
                    preferred_element_type=jnp.float32) + b2_ref[0, 0][None, :]
        y_ref[...] = y.astype(jnp.bfloat16)


def _ffn_b(hid, W2, b2, meta):
    grid_spec = pltpu.PrefetchScalarGridSpec(
        num_scalar_prefetch=1,
        grid=(R,),
        in_specs=[
            pl.BlockSpec((BLK, H), lambda r, m: (jnp.minimum(r, m[R] - 1), 0)),
            pl.BlockSpec((1, H, D), lambda r, m: (m[r], 0, 0)),
            pl.BlockSpec((1, 1, D), lambda r, m: (m[r], 0, 0)),
        ],
        out_specs=pl.BlockSpec(
            (BLK, D), lambda r, m: (jnp.minimum(r, m[R] - 1), 0)),
    )
    return pl.pallas_call(
        _ffn_b_kernel,
        grid_spec=grid_spec,
        out_shape=jax.ShapeDtypeStruct((P, D), jnp.bfloat16),
    )(meta, hid, W2, b2.reshape(E, 1, D))


def _pair_add_kernel(ya_ref, yb_ref, w_ref, out_ref):
    wa = w_ref[:, 0:1]
    wb = w_ref[:, 1:2]
    out_ref[...] = (ya_ref[...].astype(jnp.float32) * wa
                    + yb_ref[...].astype(jnp.float32) * wb)


def _pair_add(y_pairs, w_out):
    tb = 256
    nb = N // tb
    return pl.pallas_call(
        _pair_add_kernel,
        grid=(nb,),
        in_specs=[
            pl.BlockSpec((tb, D), lambda r: (r, 0)),
            pl.BlockSpec((tb, D), lambda r: (r + nb, 0)),
            pl.BlockSpec((tb, LANES), lambda r: (r, 0)),
        ],
        out_specs=pl.BlockSpec((tb, D), lambda r: (r, 0)),
        out_shape=jax.ShapeDtypeStruct((N, D), jnp.float32),
    )(y_pairs, y_pairs, w_out)


def kernel(x, gate_W, gate_b, W1, b1, W2, b2):
    gate_logits = x @ gate_W + gate_b
    idx_out, w_out, stats, loss_out, rank_out = _gating(gate_logits)
    top_idx = idx_out[:, :K]
    loss = loss_out[0, 0]

    counts = stats[1, :E].astype(jnp.int32)
    pad_counts = ((counts + BLK - 1) // BLK) * BLK
    csum_pad = jnp.cumsum(pad_counts)
    pad_off = csum_pad - pad_counts
    slot1 = jnp.take(pad_off, top_idx[:, 0]) + rank_out[:, 0]
    slot2 = jnp.take(pad_off, top_idx[:, 1]) + rank_out[:, 1]
    slot_all = jnp.concatenate([slot1, slot2])
    tok = jnp.arange(N, dtype=jnp.int32)
    init = jnp.arange(P, dtype=jnp.int32) % N
    tok_pad = init.at[slot_all].set(jnp.concatenate([tok, tok]))
    n_active = (csum_pad[E - 1] // BLK).astype(jnp.int32)
    rblk = jnp.arange(R, dtype=jnp.int32)
    blk_e = jnp.sum(
        (csum_pad[None, :] <= (rblk * BLK)[:, None]).astype(jnp.int32),
        axis=1)
    blk_e = jnp.minimum(blk_e, E - 1)
    last_e = jnp.take(blk_e, n_active - 1)
    blk_e = jnp.where(rblk < n_active, blk_e, last_e)
    meta = jnp.concatenate([blk_e, n_active[None]])

    x_pad = _sc_gather(x, tok_pad)
    hid = _ffn_a(x_pad, W1, b1, meta)
    y_pad = _ffn_b(hid, W2, b2, meta)
    y_bits = lax.bitcast_convert_type(
        y_pad.reshape(P, D // 2, 2), jnp.float32)
    g_bits = _sc_gather(y_bits, slot_all)
    y_pairs = lax.bitcast_convert_type(
        g_bits, jnp.bfloat16).reshape(2 * N, D)
    out = _pair_add(y_pairs, w_out)
    return (out, loss)


# speedup vs baseline: 1.7710x; 1.7710x over previous
import functools

import jax
import jax.numpy as jnp
from jax import lax
from jax.experimental import pallas as pl
from jax.experimental.pallas import tpu as pltpu
from jax.experimental.pallas import tpu_sc as plsc

N = 4096
D = 1024
H = 4096
E = 16
K = 2
LANES = 128
BLK = 256
P = N * K + E * BLK
R = P // BLK
GB = 512


def _gating_kernel(lg_ref, idx_ref, w_ref, stats_ref, loss_ref, rank_ref):
    r = pl.program_id(0)
    nblk = pl.num_programs(0)
    logits = lg_ref[...]
    lane = lax.broadcasted_iota(jnp.int32, (GB, LANES), 1)
    valid = lane < E
    neg = jnp.float32(-1e30)
    lg = jnp.where(valid, logits, neg)
    m1 = jnp.max(lg, axis=1, keepdims=True)
    is1 = (lg == m1) & valid
    a1 = jnp.min(jnp.where(is1, lane, LANES), axis=1, keepdims=True)
    lg2 = jnp.where(lane == a1, neg, lg)
    m2 = jnp.max(lg2, axis=1, keepdims=True)
    is2 = (lg2 == m2) & valid
    a2 = jnp.min(jnp.where(is2, lane, LANES), axis=1, keepdims=True)
    w1 = 1.0 / (1.0 + jnp.exp(m2 - m1))
    w2 = 1.0 - w1
    idx_ref[...] = jnp.where(lane == 0, a1, jnp.where(lane == 1, a2, 0))
    w_ref[...] = jnp.where(lane == 0, w1, jnp.where(lane == 1, w2, 0.0))
    p = jnp.where(valid, jnp.exp(lg - m1), 0.0)
    p = p / jnp.sum(p, axis=1, keepdims=True)
    onehot = ((lane == a1) | (lane == a2)).astype(jnp.float32)

    @pl.when(r == 0)
    def _():
        stats_ref[...] = jnp.zeros_like(stats_ref)

    oh1 = (lane == a1).astype(jnp.float32)
    oh2 = (lane == a2).astype(jnp.float32)
    ri = lax.broadcasted_iota(jnp.int32, (GB, GB), 0)
    ci = lax.broadcasted_iota(jnp.int32, (GB, GB), 1)
    lt = (ri > ci).astype(jnp.float32)
    cum_excl = jnp.dot(lt, oh1 + oh2, preferred_element_type=jnp.float32)
    base = stats_ref[1:2, :]
    rank1 = jnp.sum((cum_excl + base) * oh1, axis=1, keepdims=True)
    rank2 = jnp.sum((cum_excl + base) * oh2, axis=1, keepdims=True)
    rank_ref[...] = jnp.where(lane == 0, rank1,
                              jnp.where(lane == 1, rank2, 0.0)).astype(
                                  jnp.int32)

    stats_ref[0:1, :] += jnp.sum(p, axis=0, keepdims=True)
    stats_ref[1:2, :] += jnp.sum(onehot, axis=0, keepdims=True)

    @pl.when(r == nblk - 1)
    def _():
        prob = stats_ref[0:1, :] * (1.0 / N)
        frac = stats_ref[1:2, :] * (1.0 / N)
        lb = E * jnp.sum(prob * frac)
        loss_ref[...] = jnp.full((1, LANES), lb, jnp.float32)


def _gating(logits):
    lgp = jnp.zeros((N, LANES), jnp.float32).at[:, :E].set(logits)
    nblk = N // GB
    return pl.pallas_call(
        _gating_kernel,
        grid=(nblk,),
        in_specs=[
            pl.BlockSpec((GB, LANES), lambda r: (r, 0)),
        ],
        out_specs=[
            pl.BlockSpec((GB, LANES), lambda r: (r, 0)),
            pl.BlockSpec((GB, LANES), lambda r: (r, 0)),
            pl.BlockSpec((2, LANES), lambda r: (0, 0)),
            pl.BlockSpec((1, LANES), lambda r: (0, 0)),
            pl.BlockSpec((GB, LANES), lambda r: (r, 0)),
        ],
        out_shape=[
            jax.ShapeDtypeStruct((N, LANES), jnp.int32),
            jax.ShapeDtypeStruct((N, LANES), jnp.float32),
            jax.ShapeDtypeStruct((2, LANES), jnp.float32),
            jax.ShapeDtypeStruct((1, LANES), jnp.float32),
            jax.ShapeDtypeStruct((N, LANES), jnp.int32),
        ],
    )(lgp)


def _sc_gather(table, idx):
    n_rows = idx.shape[0]
    d = table.shape[1]
    dt = table.dtype
    nw = 32
    b_per_w = n_rows // nw
    ch = 32
    n_ch = b_per_w // ch
    mesh = plsc.VectorSubcoreMesh(core_axis_name="c", subcore_axis_name="s")

    @functools.partial(
        pl.kernel,
        mesh=mesh,
        out_type=jax.ShapeDtypeStruct((n_rows, d), dt),
        scratch_types=[
            pltpu.VMEM((b_per_w,), jnp.int32),
            pltpu.VMEM((ch, d), dt),
            pltpu.VMEM((ch, d), dt),
            pltpu.SemaphoreType.DMA,
            pltpu.SemaphoreType.DMA,
        ],
    )
    def k(table_hbm, idx_hbm, out_hbm, idx_v, buf0, buf1, sem0, sem1):
        wid = lax.axis_index("s") * 2 + lax.axis_index("c")
        base = wid * b_per_w
        bufs = (buf0, buf1)
        sems = (sem0, sem1)
        pltpu.sync_copy(idx_hbm.at[pl.ds(base, b_per_w)], idx_v)
        pend = [None, None]
        pend[0] = pltpu.async_copy(
            table_hbm.at[idx_v.at[pl.ds(0, ch)]], bufs[0], sems[0])
        for c in range(n_ch):
            b = c % 2
            if c + 1 < n_ch:
                pend[1 - b] = pltpu.async_copy(
                    table_hbm.at[idx_v.at[pl.ds((c + 1) * ch, ch)]],
                    bufs[1 - b], sems[1 - b])
            pend[b].wait()
            pltpu.sync_copy(bufs[b], out_hbm.at[pl.ds(base + c * ch, ch)])

    return k(table, idx)


def _ffn_a_kernel(meta_ref, x_ref, w1_ref, b1_ref, h_ref):
    r = pl.program_id(0)

    @pl.when(r < meta_ref[R])
    def _():
        h = jnp.dot(x_ref[...], w1_ref[0],
                    preferred_element_type=jnp.float32) + b1_ref[0, 0][None, :]
        h_ref[...] = jnp.maximum(h, 0.0).astype(jnp.bfloat16)


def _ffn_a(x_pad, W1, b1, meta):
    grid_spec = pltpu.PrefetchScalarGridSpec(
        num_scalar_prefetch=1,
        grid=(R,),
        in_specs=[
            pl.BlockSpec((BLK, D), lambda r, m: (jnp.minimum(r, m[R] - 1), 0)),
            pl.BlockSpec((1, D, H), lambda r, m: (m[r], 0, 0)),
            pl.BlockSpec((1, 1, H), lambda r, m: (m[r], 0, 0)),
        ],
        out_specs=pl.BlockSpec(
            (BLK, H), lambda r, m: (jnp.minimum(r, m[R] - 1), 0)),
    )
    return pl.pallas_call(
        _ffn_a_kernel,
        grid_spec=grid_spec,
        out_shape=jax.ShapeDtypeStruct((P, H), jnp.bfloat16),
    )(meta, x_pad, W1, b1.reshape(E, 1, H))


def _ffn_b_kernel(meta_ref, h_ref, w2_ref, b2_ref, y_ref):
    r = pl.program_id(0)

    @pl.when(r < meta_ref[R])
    def _():
        y = jnp.dot(h_ref[...].astype(jnp.float32), w2_ref[0],
                    preferred_element_type=jnp.float32) + b2_ref[0, 0][None, :]
        y_ref[...] = y


def _ffn_b(hid, W2, b2, meta):
    grid_spec = pltpu.PrefetchScalarGridSpec(
        num_scalar_prefetch=1,
        grid=(R,),
        in_specs=[
            pl.BlockSpec((BLK, H), lambda r, m: (jnp.minimum(r, m[R] - 1), 0)),
            pl.BlockSpec((1, H, D), lambda r, m: (m[r], 0, 0)),
            pl.BlockSpec((1, 1, D), lambda r, m: (m[r], 0, 0)),
        ],
        out_specs=pl.BlockSpec(
            (BLK, D), lambda r, m: (jnp.minimum(r, m[R] - 1), 0)),
    )
    return pl.pallas_call(
        _ffn_b_kernel,
        grid_spec=grid_spec,
        out_shape=jax.ShapeDtypeStruct((P, D), jnp.float32),
    )(meta, hid, W2, b2.reshape(E, 1, D))


def _pair_add_kernel(ya_ref, yb_ref, w_ref, out_ref):
    wa = w_ref[:, 0:1]
    wb = w_ref[:, 1:2]
    out_ref[...] = (ya_ref[...].astype(jnp.float32) * wa
                    + yb_ref[...].astype(jnp.float32) * wb)


def _pair_add(y_pairs, w_out):
    tb = 256
    nb = N // tb
    return pl.pallas_call(
        _pair_add_kernel,
        grid=(nb,),
        in_specs=[
            pl.BlockSpec((tb, D), lambda r: (r, 0)),
            pl.BlockSpec((tb, D), lambda r: (r + nb, 0)),
            pl.BlockSpec((tb, LANES), lambda r: (r, 0)),
        ],
        out_specs=pl.BlockSpec((tb, D), lambda r: (r, 0)),
        out_shape=jax.ShapeDtypeStruct((N, D), jnp.float32),
    )(y_pairs, y_pairs, w_out)


def kernel(x, gate_W, gate_b, W1, b1, W2, b2):
    gate_logits = x @ gate_W + gate_b
    idx_out, w_out, stats, loss_out, rank_out = _gating(gate_logits)
    top_idx = idx_out[:, :K]
    loss = loss_out[0, 0]

    counts = stats[1, :E].astype(jnp.int32)
    pad_counts = ((counts + BLK - 1) // BLK) * BLK
    csum_pad = jnp.cumsum(pad_counts)
    pad_off = csum_pad - pad_counts
    slot1 = jnp.take(pad_off, top_idx[:, 0]) + rank_out[:, 0]
    slot2 = jnp.take(pad_off, top_idx[:, 1]) + rank_out[:, 1]
    slot_all = jnp.concatenate([slot1, slot2])
    tok = jnp.arange(N, dtype=jnp.int32)
    init = jnp.arange(P, dtype=jnp.int32) % N
    tok_pad = init.at[slot_all].set(jnp.concatenate([tok, tok]))
    n_active = (csum_pad[E - 1] // BLK).astype(jnp.int32)
    rblk = jnp.arange(R, dtype=jnp.int32)
    blk_e = jnp.sum(
        (csum_pad[None, :] <= (rblk * BLK)[:, None]).astype(jnp.int32),
        axis=1)
    blk_e = jnp.minimum(blk_e, E - 1)
    last_e = jnp.take(blk_e, n_active - 1)
    blk_e = jnp.where(rblk < n_active, blk_e, last_e)
    meta = jnp.concatenate([blk_e, n_active[None]])

    x_pad = _sc_gather(x, tok_pad)
    hid = _ffn_a(x_pad, W1, b1, meta)
    y_pad = _ffn_b(hid, W2, b2, meta)
    y_pairs = _sc_gather(y_pad, slot_all)
    out = _pair_add(y_pairs, w_out)
    return (out, loss)
